# baseline (speedup 1.0000x reference)
import functools

import jax
import jax.numpy as jnp
from jax import lax
from jax.experimental import pallas as pl
from jax.experimental.pallas import tpu as pltpu
from jax.experimental.pallas import tpu_sc as plsc

N = 10000
D = 128
DEG = 32
E = N * DEG

NC = 2
NS = 16
NW = NC * NS

CH = 80
NCH = N // CH

B = 2000
P = N // B


D2 = D // 2

XLOAD_W = 10
XLOAD_R = N // XLOAD_W


KSEG = DEG // 2
HCH0 = (NCH + 1) // 2


def _sc_build_mailboxes(x, gidx, sidx):
    mesh = plsc.VectorSubcoreMesh(core_axis_name="c", subcore_axis_name="s")

    @functools.partial(
        pl.kernel,
        out_type=(
            jax.ShapeDtypeStruct((KSEG * N, D), jnp.float32),
            jax.ShapeDtypeStruct((KSEG * N, D), jnp.float32),
        ),
        mesh=mesh,
        scratch_types=[
            pltpu.VMEM_SHARED((N, D), jnp.float32),
            pltpu.VMEM((4, CH), jnp.int32),
            pltpu.VMEM((4, CH), jnp.int32),
            pltpu.VMEM((2, CH, D), jnp.float32),
            pltpu.VMEM((2, CH, D), jnp.float32),
            pltpu.SemaphoreType.DMA,
            pltpu.SemaphoreType.DMA,
            pltpu.SemaphoreType.DMA,
            pltpu.SemaphoreType.DMA((2,)),
            pltpu.SemaphoreType.DMA((2,)),
        ],
    )
    def sc_kernel(x_hbm, gidx_hbm, sidx_hbm, mail1_hbm, mail2_hbm,
                  x_sh, gi_c, si_c, buf1, buf2,
                  sem_i, sem_g1, sem_g2, sem_w1, sem_w2):
        s = lax.axis_index("s")
        w = s * NC + lax.axis_index("c")
        seg = w // 2
        half = w % 2
        base = seg * N
        lo = half * HCH0
        hi = lo + HCH0 - half * (2 * HCH0 - NCH)

        @pl.when(s < XLOAD_W)
        def _():
            pltpu.sync_copy(x_hbm.at[pl.ds(s * XLOAD_R, XLOAD_R)],
                            x_sh.at[pl.ds(s * XLOAD_R, XLOAD_R)])

        def fire_idx(i):
            b = lax.rem(i, 4)
            pltpu.async_copy(gidx_hbm.at[seg].at[i], gi_c.at[b], sem_i)
            pltpu.async_copy(sidx_hbm.at[seg].at[i], si_c.at[b], sem_i)

        def wait_idx(i):
            b = lax.rem(i, 4)
            pltpu.make_async_copy(gidx_hbm.at[0].at[0], gi_c.at[b], sem_i).wait()
            pltpu.make_async_copy(gidx_hbm.at[0].at[0], si_c.at[b], sem_i).wait()

        def wait_w(i):
            b = lax.rem(i, 2)
            pltpu.make_async_copy(
                buf1.at[0], mail1_hbm.at[pl.ds(base + i * CH, CH)],
                sem_w1.at[b]).wait()
            pltpu.make_async_copy(
                buf2.at[0], mail2_hbm.at[pl.ds(base + i * CH, CH)],
                sem_w2.at[b]).wait()

        fire_idx(lo)
        plsc.subcore_barrier()

        def body(i, carry):
            b = lax.rem(i, 2)

            @pl.when(i >= lo + 2)
            def _():
                wait_w(i - 2)

            wait_idx(i)

            @pl.when(i + 1 < hi)
            def _():
                fire_idx(i + 1)

            pltpu.async_copy(x_sh.at[gi_c.at[lax.rem(i, 4)]], buf1.at[b], sem_g1)
            pltpu.async_copy(x_sh.at[pl.ds(i * CH, CH)], buf2.at[b], sem_g2)
            pltpu.make_async_copy(x_hbm.at[pl.ds(0, CH)], buf1.at[b], sem_g1).wait()
            pltpu.make_async_copy(x_hbm.at[pl.ds(0, CH)], buf2.at[b], sem_g2).wait()

            pltpu.async_copy(buf1.at[b],
                             mail1_hbm.at[pl.ds(base + i * CH, CH)], sem_w1.at[b])
            pltpu.async_copy(buf2.at[b], mail2_hbm.at[si_c.at[lax.rem(i, 4)]],
                             sem_w2.at[b])
            return carry

        lax.fori_loop(lo, hi, body, 0)
        wait_w(hi - 2)
        wait_w(hi - 1)

    return sc_kernel(x, gidx, sidx)


def _mean_body(x_ref, o_ref):
    i = pl.program_id(0)

    @pl.when(i == 0)
    def _():
        o_ref[...] = jnp.zeros_like(o_ref)

    o_ref[...] += jnp.sum(x_ref[...], axis=0, keepdims=True) * (1.0 / N)


def _tc_mean(x):
    return pl.pallas_call(
        _mean_body,
        grid=(P,),
        in_specs=[pl.BlockSpec((B, D), lambda i: (i, 0))],
        out_specs=pl.BlockSpec((1, D), lambda i: (0, 0)),
        out_shape=jax.ShapeDtypeStruct((1, D), jnp.float32),
    )(x)


def _cell(a_cur, h_ref, c_ref, whh_ref, bi_ref):
    hb = h_ref[...].astype(jnp.bfloat16)
    gates = (a_cur
             + jnp.dot(hb, whh_ref[...], preferred_element_type=jnp.float32)
             + bi_ref[...])
    ii = gates[:, 0:D]
    ff = gates[:, D:2 * D]
    gg = gates[:, 2 * D:3 * D]
    oo = gates[:, 3 * D:4 * D]
    c = jax.nn.sigmoid(ff) * c_ref[...] + jax.nn.sigmoid(ii) * jnp.tanh(gg)
    h = jax.nn.sigmoid(oo) * jnp.tanh(c)
    h_ref[...] = h
    c_ref[...] = c


def _conv_step(k, m0_ref, mn_ref, wih_ref, whh_ref, bi_ref, h, c, A):
    @pl.when(k == 0)
    def _():
        A[...] = jnp.dot(m0_ref[0].astype(jnp.bfloat16), wih_ref[...],
                         preferred_element_type=jnp.float32)

    _cell(A[...], h, c, whh_ref, bi_ref)

    @pl.when(k < KSEG - 1)
    def _():
        A[...] = jnp.dot(mn_ref[0].astype(jnp.bfloat16), wih_ref[...],
                         preferred_element_type=jnp.float32)


def _lstm_body_a(m01_ref, m02_ref, m1_ref, m2_ref,
                 wih1_ref, whh1_ref, wih2_ref, whh2_ref,
                 bi1_ref, bi2_ref,
                 h1o_ref, c1o_ref, h2o_ref, c2o_ref,
                 h1, c1, h2, c2, A1, A2):
    k = pl.program_id(1)

    @pl.when(k == 0)
    def _():
        h1[...] = jnp.zeros_like(h1)
        c1[...] = jnp.zeros_like(c1)
        h2[...] = jnp.zeros_like(h2)
        c2[...] = jnp.zeros_like(c2)

    _conv_step(k, m01_ref, m1_ref, wih1_ref, whh1_ref, bi1_ref, h1, c1, A1)
    _conv_step(k, m02_ref, m2_ref, wih2_ref, whh2_ref, bi2_ref, h2, c2, A2)

    @pl.when(k == KSEG - 1)
    def _():
        h1o_ref[...] = h1[...]
        c1o_ref[...] = c1[...]
        h2o_ref[...] = h2[...]
        c2o_ref[...] = c2[...]


def _lstm_body_b(m01_ref, m02_ref, m1_ref, m2_ref,
                 h1i_ref, c1i_ref, h2i_ref, c2i_ref, x_ref,
                 wih1_ref, whh1_ref, wih2_ref, whh2_ref,
                 fcn1_ref, fcn2_ref, fcs_ref, bi1_ref, bi2_ref, boff_ref,
                 o_ref, h1, c1, h2, c2, A1, A2):
    k = pl.program_id(1)

    @pl.when(k == 0)
    def _():
        h1[...] = h1i_ref[...]
        c1[...] = c1i_ref[...]
        h2[...] = h2i_ref[...]
        c2[...] = c2i_ref[...]

    _conv_step(k, m01_ref, m1_ref, wih1_ref, whh1_ref, bi1_ref, h1, c1, A1)
    _conv_step(k, m02_ref, m2_ref, wih2_ref, whh2_ref, bi2_ref, h2, c2, A2)

    @pl.when(k == KSEG - 1)
    def _():
        o_ref[...] = (jnp.dot(x_ref[...], fcs_ref[...],
                              preferred_element_type=jnp.float32)
                      + jnp.dot(h1[...], fcn1_ref[...],
                                preferred_element_type=jnp.float32)
                      + jnp.dot(h2[...], fcn2_ref[...],
                                preferred_element_type=jnp.float32)
                      + boff_ref[...])


def _full(shape):
    return pl.BlockSpec(shape, lambda nb, k: tuple(0 for _ in shape))


_MAIL0_SPEC = pl.BlockSpec((1, B, D), lambda nb, k: (0, nb, 0))
_MAILN_SPEC = pl.BlockSpec(
    (1, B, D), lambda nb, k: (jnp.minimum(k + 1, KSEG - 1), nb, 0))
_ST_SPEC = pl.BlockSpec((B, D), lambda nb, k: (nb, 0))
_STATE = jax.ShapeDtypeStruct((N, D), jnp.float32)


def _tc_lstm_a(mail1, mail2, wih1T, whh1T, wih2T, whh2T, bi1, bi2):
    return pl.pallas_call(
        _lstm_body_a,
        grid=(P, KSEG),
        in_specs=[
            _MAIL0_SPEC, _MAIL0_SPEC, _MAILN_SPEC, _MAILN_SPEC,
            _full((D, 4 * D)), _full((D, 4 * D)),
            _full((D, 4 * D)), _full((D, 4 * D)),
            _full((1, 4 * D)), _full((1, 4 * D)),
        ],
        out_specs=[_ST_SPEC, _ST_SPEC, _ST_SPEC, _ST_SPEC],
        out_shape=[_STATE, _STATE, _STATE, _STATE],
        scratch_shapes=[pltpu.VMEM((B, D), jnp.float32)] * 4
        + [pltpu.VMEM((B, 4 * D), jnp.float32)] * 2,
    )(mail1, mail2, mail1, mail2, wih1T, whh1T, wih2T, whh2T, bi1, bi2)


def _tc_lstm_b(mail1, mail2, st, x, wih1T, whh1T, wih2T, whh2T,
               fcn1T, fcn2T, fcsT, bi1, bi2, boff):
    return pl.pallas_call(
        _lstm_body_b,
        grid=(P, KSEG),
        in_specs=[
            _MAIL0_SPEC, _MAIL0_SPEC, _MAILN_SPEC, _MAILN_SPEC,
            _ST_SPEC, _ST_SPEC, _ST_SPEC, _ST_SPEC,
            _ST_SPEC,
            _full((D, 4 * D)), _full((D, 4 * D)),
            _full((D, 4 * D)), _full((D, 4 * D)),
            _full((D, D)), _full((D, D)), _full((D, D)),
            _full((1, 4 * D)), _full((1, 4 * D)), _full((1, D)),
        ],
        out_specs=_ST_SPEC,
        out_shape=jax.ShapeDtypeStruct((N, D), jnp.float32),
        scratch_shapes=[pltpu.VMEM((B, D), jnp.float32)] * 4
        + [pltpu.VMEM((B, 4 * D), jnp.float32)] * 2,
    )(mail1, mail2, mail1, mail2, *st, x, wih1T, whh1T, wih2T, whh2T,
      fcn1T, fcn2T, fcsT, bi1, bi2, boff)


def kernel(x, edge_index, fc_self1, fc_neigh1, bias1,
           lstm1_Wih, lstm1_Whh, lstm1_bih, lstm1_bhh,
           fc_self2, fc_neigh2, bias2,
           lstm2_Wih, lstm2_Whh, lstm2_bih, lstm2_bhh):
    src = edge_index[0].astype(jnp.int32)
    seg_off = jnp.repeat(jnp.arange(DEG, dtype=jnp.int32) % KSEG * N, N)
    gidx = src.reshape(DEG, NCH, CH)
    sidx = (src + seg_off).reshape(DEG, NCH, CH)

    xmean = _tc_mean(x)

    wih1T = lstm1_Wih.T.astype(jnp.bfloat16)
    whh1T = lstm1_Whh.T.astype(jnp.bfloat16)
    wih2T = lstm2_Wih.T.astype(jnp.bfloat16)
    whh2T = lstm2_Whh.T.astype(jnp.bfloat16)
    bi1 = (lstm1_bih + lstm1_bhh).reshape(1, 4 * D)
    bi2 = (lstm2_bih + lstm2_bhh).reshape(1, 4 * D)
    fcn1T = fc_neigh1.T
    fcn2T = fc_neigh2.T
    fcsT = (fc_self1 + fc_self2).T
    boff = (bias1 + bias2).reshape(1, D) + xmean

    m1a, m2a = _sc_build_mailboxes(x, gidx[:KSEG], sidx[:KSEG])
    m1b, m2b = _sc_build_mailboxes(x, gidx[KSEG:], sidx[KSEG:])

    st = _tc_lstm_a(m1a.reshape(KSEG, N, D), m2a.reshape(KSEG, N, D),
                    wih1T, whh1T, wih2T, whh2T, bi1, bi2)

    return _tc_lstm_b(m1b.reshape(KSEG, N, D), m2b.reshape(KSEG, N, D),
                      st, x, wih1T, whh1T, wih2T, whh2T,
                      fcn1T, fcn2T, fcsT, bi1, bi2, boff)


# speedup vs baseline: 1.1873x; 1.1873x over previous
import functools

import jax
import jax.numpy as jnp
from jax import lax
from jax.experimental import pallas as pl
from jax.experimental.pallas import tpu as pltpu
from jax.experimental.pallas import tpu_sc as plsc

N = 10000
D = 128
DEG = 32
E = N * DEG

NC = 2
NS = 16
NW = NC * NS

CH = 80
NCH = N // CH

B = 5000
P = N // B
BB = 2000
PB = N // BB


D2 = D // 2

XLOAD_W = 10
XLOAD_R = N // XLOAD_W


KSEG = DEG // 2
HCH0 = (NCH + 1) // 2


def _sc_build_mailboxes(x, gidx, sidx):
    mesh = plsc.VectorSubcoreMesh(core_axis_name="c", subcore_axis_name="s")

    @functools.partial(
        pl.kernel,
        out_type=(
            jax.ShapeDtypeStruct((KSEG * N, D), jnp.float32),
            jax.ShapeDtypeStruct((KSEG * N, D), jnp.float32),
        ),
        mesh=mesh,
        scratch_types=[
            pltpu.VMEM_SHARED((N, D), jnp.float32),
            pltpu.VMEM((4, CH), jnp.int32),
            pltpu.VMEM((4, CH), jnp.int32),
            pltpu.VMEM((2, CH, D), jnp.float32),
            pltpu.VMEM((2, CH, D), jnp.float32),
            pltpu.SemaphoreType.DMA,
            pltpu.SemaphoreType.DMA,
            pltpu.SemaphoreType.DMA,
            pltpu.SemaphoreType.DMA((2,)),
            pltpu.SemaphoreType.DMA((2,)),
        ],
    )
    def sc_kernel(x_hbm, gidx_hbm, sidx_hbm, mail1_hbm, mail2_hbm,
                  x_sh, gi_c, si_c, buf1, buf2,
                  sem_i, sem_g1, sem_g2, sem_w1, sem_w2):
        s = lax.axis_index("s")
        w = s * NC + lax.axis_index("c")
        seg = w // 2
        half = w % 2
        base = seg * N
        lo = half * HCH0
        hi = lo + HCH0 - half * (2 * HCH0 - NCH)

        @pl.when(s < XLOAD_W)
        def _():
            pltpu.sync_copy(x_hbm.at[pl.ds(s * XLOAD_R, XLOAD_R)],
                            x_sh.at[pl.ds(s * XLOAD_R, XLOAD_R)])

        def fire_idx(i):
            b = lax.rem(i, 4)
            pltpu.async_copy(gidx_hbm.at[seg].at[i], gi_c.at[b], sem_i)
            pltpu.async_copy(sidx_hbm.at[seg].at[i], si_c.at[b], sem_i)

        def wait_idx(i):
            b = lax.rem(i, 4)
            pltpu.make_async_copy(gidx_hbm.at[0].at[0], gi_c.at[b], sem_i).wait()
            pltpu.make_async_copy(gidx_hbm.at[0].at[0], si_c.at[b], sem_i).wait()

        def wait_w(i):
            b = lax.rem(i, 2)
            pltpu.make_async_copy(
                buf1.at[0], mail1_hbm.at[pl.ds(base + i * CH, CH)],
                sem_w1.at[b]).wait()
            pltpu.make_async_copy(
                buf2.at[0], mail2_hbm.at[pl.ds(base + i * CH, CH)],
                sem_w2.at[b]).wait()

        fire_idx(lo)
        plsc.subcore_barrier()

        def body(i, carry):
            b = lax.rem(i, 2)

            @pl.when(i >= lo + 2)
            def _():
                wait_w(i - 2)

            wait_idx(i)

            @pl.when(i + 1 < hi)
            def _():
                fire_idx(i + 1)

            pltpu.async_copy(x_sh.at[gi_c.at[lax.rem(i, 4)]], buf1.at[b], sem_g1)
            pltpu.async_copy(x_sh.at[pl.ds(i * CH, CH)], buf2.at[b], sem_g2)
            pltpu.make_async_copy(x_hbm.at[pl.ds(0, CH)], buf1.at[b], sem_g1).wait()
            pltpu.make_async_copy(x_hbm.at[pl.ds(0, CH)], buf2.at[b], sem_g2).wait()

            pltpu.async_copy(buf1.at[b],
                             mail1_hbm.at[pl.ds(base + i * CH, CH)], sem_w1.at[b])
            pltpu.async_copy(buf2.at[b], mail2_hbm.at[si_c.at[lax.rem(i, 4)]],
                             sem_w2.at[b])
            return carry

        lax.fori_loop(lo, hi, body, 0)
        wait_w(hi - 2)
        wait_w(hi - 1)

    return sc_kernel(x, gidx, sidx)


def _mean_body(x_ref, o_ref):
    i = pl.program_id(0)

    @pl.when(i == 0)
    def _():
        o_ref[...] = jnp.zeros_like(o_ref)

    o_ref[...] += jnp.sum(x_ref[...], axis=0, keepdims=True) * (1.0 / N)


def _tc_mean(x):
    return pl.pallas_call(
        _mean_body,
        grid=(P,),
        in_specs=[pl.BlockSpec((B, D), lambda i: (i, 0))],
        out_specs=pl.BlockSpec((1, D), lambda i: (0, 0)),
        out_shape=jax.ShapeDtypeStruct((1, D), jnp.float32),
    )(x)


def _cell(m_ref, h_ref, c_ref, wih_ref, whh_ref, bi_ref):
    hb = h_ref[...].astype(jnp.bfloat16)
    mb = m_ref[0].astype(jnp.bfloat16)
    gates = (jnp.dot(mb, wih_ref[...], preferred_element_type=jnp.float32)
             + jnp.dot(hb, whh_ref[...], preferred_element_type=jnp.float32)
             + bi_ref[...])
    ii = gates[:, 0:D]
    ff = gates[:, D:2 * D]
    gg = gates[:, 2 * D:3 * D]
    oo = gates[:, 3 * D:4 * D]
    c = jax.nn.sigmoid(ff) * c_ref[...] + jax.nn.sigmoid(ii) * jnp.tanh(gg)
    h = jax.nn.sigmoid(oo) * jnp.tanh(c)
    h_ref[...] = h
    c_ref[...] = c


def _lstm_body_a(m1_ref, m2_ref, wih1_ref, whh1_ref, wih2_ref, whh2_ref,
                 bi1_ref, bi2_ref,
                 h1o_ref, c1o_ref, h2o_ref, c2o_ref, h1, c1, h2, c2):
    k = pl.program_id(1)

    @pl.when(k == 0)
    def _():
        h1[...] = jnp.zeros_like(h1)
        c1[...] = jnp.zeros_like(c1)
        h2[...] = jnp.zeros_like(h2)
        c2[...] = jnp.zeros_like(c2)

    _cell(m1_ref, h1, c1, wih1_ref, whh1_ref, bi1_ref)
    _cell(m2_ref, h2, c2, wih2_ref, whh2_ref, bi2_ref)

    @pl.when(k == KSEG - 1)
    def _():
        h1o_ref[...] = h1[...]
        c1o_ref[...] = c1[...]
        h2o_ref[...] = h2[...]
        c2o_ref[...] = c2[...]


def _lstm_body_b(m1_ref, m2_ref, h1i_ref, c1i_ref, h2i_ref, c2i_ref, x_ref,
                 wih1_ref, whh1_ref, wih2_ref, whh2_ref,
                 fcn1_ref, fcn2_ref, fcs_ref, bi1_ref, bi2_ref, boff_ref,
                 o_ref, h1, c1, h2, c2):
    k = pl.program_id(1)

    @pl.when(k == 0)
    def _():
        h1[...] = h1i_ref[...]
        c1[...] = c1i_ref[...]
        h2[...] = h2i_ref[...]
        c2[...] = c2i_ref[...]

    _cell(m1_ref, h1, c1, wih1_ref, whh1_ref, bi1_ref)
    _cell(m2_ref, h2, c2, wih2_ref, whh2_ref, bi2_ref)

    @pl.when(k == KSEG - 1)
    def _():
        o_ref[...] = (jnp.dot(x_ref[...], fcs_ref[...],
                              preferred_element_type=jnp.float32)
                      + jnp.dot(h1[...], fcn1_ref[...],
                                preferred_element_type=jnp.float32)
                      + jnp.dot(h2[...], fcn2_ref[...],
                                preferred_element_type=jnp.float32)
                      + boff_ref[...])


def _full(shape):
    return pl.BlockSpec(shape, lambda nb, k: tuple(0 for _ in shape))


_MAIL_SPEC = pl.BlockSpec((1, B, D), lambda nb, k: (k, nb, 0))
_ST_SPEC = pl.BlockSpec((B, D), lambda nb, k: (nb, 0))
_MAIL_SPEC_B = pl.BlockSpec((1, BB, D), lambda nb, k: (k, nb, 0))
_ST_SPEC_B = pl.BlockSpec((BB, D), lambda nb, k: (nb, 0))
_STATE = jax.ShapeDtypeStruct((N, D), jnp.float32)


def _tc_lstm_a(mail1, mail2, wih1T, whh1T, wih2T, whh2T, bi1, bi2):
    return pl.pallas_call(
        _lstm_body_a,
        grid=(P, KSEG),
        in_specs=[
            _MAIL_SPEC, _MAIL_SPEC,
            _full((D, 4 * D)), _full((D, 4 * D)),
            _full((D, 4 * D)), _full((D, 4 * D)),
            _full((1, 4 * D)), _full((1, 4 * D)),
        ],
        out_specs=[_ST_SPEC, _ST_SPEC, _ST_SPEC, _ST_SPEC],
        out_shape=[_STATE, _STATE, _STATE, _STATE],
        scratch_shapes=[pltpu.VMEM((B, D), jnp.float32)] * 4,
    )(mail1, mail2, wih1T, whh1T, wih2T, whh2T, bi1, bi2)


def _tc_lstm_b(mail1, mail2, st, x, wih1T, whh1T, wih2T, whh2T,
               fcn1T, fcn2T, fcsT, bi1, bi2, boff):
    return pl.pallas_call(
        _lstm_body_b,
        grid=(PB, KSEG),
        in_specs=[
            _MAIL_SPEC_B, _MAIL_SPEC_B,
            _ST_SPEC_B, _ST_SPEC_B, _ST_SPEC_B, _ST_SPEC_B,
            _ST_SPEC_B,
            _full((D, 4 * D)), _full((D, 4 * D)),
            _full((D, 4 * D)), _full((D, 4 * D)),
            _full((D, D)), _full((D, D)), _full((D, D)),
            _full((1, 4 * D)), _full((1, 4 * D)), _full((1, D)),
        ],
        out_specs=_ST_SPEC_B,
        out_shape=jax.ShapeDtypeStruct((N, D), jnp.float32),
        scratch_shapes=[pltpu.VMEM((BB, D), jnp.float32)] * 4,
    )(mail1, mail2, *st, x, wih1T, whh1T, wih2T, whh2T,
      fcn1T, fcn2T, fcsT, bi1, bi2, boff)


def kernel(x, edge_index, fc_self1, fc_neigh1, bias1,
           lstm1_Wih, lstm1_Whh, lstm1_bih, lstm1_bhh,
           fc_self2, fc_neigh2, bias2,
           lstm2_Wih, lstm2_Whh, lstm2_bih, lstm2_bhh):
    src = edge_index[0].astype(jnp.int32)
    seg_off = jnp.repeat(jnp.arange(DEG, dtype=jnp.int32) % KSEG * N, N)
    gidx = src.reshape(DEG, NCH, CH)
    sidx = (src + seg_off).reshape(DEG, NCH, CH)

    xmean = _tc_mean(x)

    wih1T = lstm1_Wih.T.astype(jnp.bfloat16)
    whh1T = lstm1_Whh.T.astype(jnp.bfloat16)
    wih2T = lstm2_Wih.T.astype(jnp.bfloat16)
    whh2T = lstm2_Whh.T.astype(jnp.bfloat16)
    bi1 = (lstm1_bih + lstm1_bhh).reshape(1, 4 * D)
    bi2 = (lstm2_bih + lstm2_bhh).reshape(1, 4 * D)
    fcn1T = fc_neigh1.T
    fcn2T = fc_neigh2.T
    fcsT = (fc_self1 + fc_self2).T
    boff = (bias1 + bias2).reshape(1, D) + xmean

    m1a, m2a = _sc_build_mailboxes(x, gidx[:KSEG], sidx[:KSEG])
    m1b, m2b = _sc_build_mailboxes(x, gidx[KSEG:], sidx[KSEG:])

    st = _tc_lstm_a(m1a.reshape(KSEG, N, D), m2a.reshape(KSEG, N, D),
                    wih1T, whh1T, wih2T, whh2T, bi1, bi2)

    return _tc_lstm_b(m1b.reshape(KSEG, N, D), m2b.reshape(KSEG, N, D),
                      st, x, wih1T, whh1T, wih2T, whh2T,
                      fcn1T, fcn2T, fcsT, bi1, bi2, boff)


# speedup vs baseline: 1.1980x; 1.0090x over previous
import functools

import jax
import jax.numpy as jnp
from jax import lax
from jax.experimental import pallas as pl
from jax.experimental.pallas import tpu as pltpu
from jax.experimental.pallas import tpu_sc as plsc

N = 10000
D = 128
DEG = 32
E = N * DEG

NC = 2
NS = 16
NW = NC * NS

CH = 80
NCH = N // CH

B = 5000
P = N // B
BB = 2000
PB = N // BB


D2 = D // 2

XLOAD_W = 10
XLOAD_R = N // XLOAD_W


KSEG = DEG // 4


def _sc_build_mailboxes(x, gidx, sidx):
    mesh = plsc.VectorSubcoreMesh(core_axis_name="c", subcore_axis_name="s")

    @functools.partial(
        pl.kernel,
        out_type=(
            jax.ShapeDtypeStruct((KSEG * N, D), jnp.float32),
            jax.ShapeDtypeStruct((KSEG * N, D), jnp.float32),
        ),
        mesh=mesh,
        scratch_types=[
            pltpu.VMEM_SHARED((N, D), jnp.float32),
            pltpu.VMEM((4, CH), jnp.int32),
            pltpu.VMEM((4, CH), jnp.int32),
            pltpu.VMEM((2, CH, D), jnp.float32),
            pltpu.VMEM((2, CH, D), jnp.float32),
            pltpu.SemaphoreType.DMA,
            pltpu.SemaphoreType.DMA,
            pltpu.SemaphoreType.DMA,
            pltpu.SemaphoreType.DMA((2,)),
            pltpu.SemaphoreType.DMA((2,)),
        ],
    )
    def sc_kernel(x_hbm, gidx_hbm, sidx_hbm, mail1_hbm, mail2_hbm,
                  x_sh, gi_c, si_c, buf1, buf2,
                  sem_i, sem_g1, sem_g2, sem_w1, sem_w2):
        s = lax.axis_index("s")
        w = s * NC + lax.axis_index("c")
        seg = w // 4
        q = w % 4
        base = seg * N
        lo = 31 * q + jnp.minimum(q, 1)
        hi = lo + 32 - jnp.minimum(q, 1)

        @pl.when(s < XLOAD_W)
        def _():
            pltpu.sync_copy(x_hbm.at[pl.ds(s * XLOAD_R, XLOAD_R)],
                            x_sh.at[pl.ds(s * XLOAD_R, XLOAD_R)])

        def fire_idx(i):
            b = lax.rem(i, 4)
            pltpu.async_copy(gidx_hbm.at[seg].at[i], gi_c.at[b], sem_i)
            pltpu.async_copy(sidx_hbm.at[seg].at[i], si_c.at[b], sem_i)

        def wait_idx(i):
            b = lax.rem(i, 4)
            pltpu.make_async_copy(gidx_hbm.at[0].at[0], gi_c.at[b], sem_i).wait()
            pltpu.make_async_copy(gidx_hbm.at[0].at[0], si_c.at[b], sem_i).wait()

        def wait_w(i):
            b = lax.rem(i, 2)
            pltpu.make_async_copy(
                buf1.at[0], mail1_hbm.at[pl.ds(base + i * CH, CH)],
                sem_w1.at[b]).wait()
            pltpu.make_async_copy(
                buf2.at[0], mail2_hbm.at[pl.ds(base + i * CH, CH)],
                sem_w2.at[b]).wait()

        fire_idx(lo)
        plsc.subcore_barrier()

        def body(i, carry):
            b = lax.rem(i, 2)

            @pl.when(i >= lo + 2)
            def _():
                wait_w(i - 2)

            wait_idx(i)

            @pl.when(i + 1 < hi)
            def _():
                fire_idx(i + 1)

            pltpu.async_copy(x_sh.at[gi_c.at[lax.rem(i, 4)]], buf1.at[b], sem_g1)
            pltpu.async_copy(x_sh.at[pl.ds(i * CH, CH)], buf2.at[b], sem_g2)
            pltpu.make_async_copy(x_hbm.at[pl.ds(0, CH)], buf1.at[b], sem_g1).wait()
            pltpu.make_async_copy(x_hbm.at[pl.ds(0, CH)], buf2.at[b], sem_g2).wait()

            pltpu.async_copy(buf1.at[b],
                             mail1_hbm.at[pl.ds(base + i * CH, CH)], sem_w1.at[b])
            pltpu.async_copy(buf2.at[b], mail2_hbm.at[si_c.at[lax.rem(i, 4)]],
                             sem_w2.at[b])
            return carry

        lax.fori_loop(lo, hi, body, 0)
        wait_w(hi - 2)
        wait_w(hi - 1)

    return sc_kernel(x, gidx, sidx)


def _mean_body(x_ref, o_ref):
    i = pl.program_id(0)

    @pl.when(i == 0)
    def _():
        o_ref[...] = jnp.zeros_like(o_ref)

    o_ref[...] += jnp.sum(x_ref[...], axis=0, keepdims=True) * (1.0 / N)


def _tc_mean(x):
    return pl.pallas_call(
        _mean_body,
        grid=(P,),
        in_specs=[pl.BlockSpec((B, D), lambda i: (i, 0))],
        out_specs=pl.BlockSpec((1, D), lambda i: (0, 0)),
        out_shape=jax.ShapeDtypeStruct((1, D), jnp.float32),
    )(x)


def _cell(m_ref, h_ref, c_ref, wih_ref, whh_ref, bi_ref):
    hb = h_ref[...].astype(jnp.bfloat16)
    mb = m_ref[0].astype(jnp.bfloat16)
    gates = (jnp.dot(mb, wih_ref[...], preferred_element_type=jnp.float32)
             + jnp.dot(hb, whh_ref[...], preferred_element_type=jnp.float32)
             + bi_ref[...])
    ii = gates[:, 0:D]
    ff = gates[:, D:2 * D]
    gg = gates[:, 2 * D:3 * D]
    oo = gates[:, 3 * D:4 * D]
    c = jax.nn.sigmoid(ff) * c_ref[...] + jax.nn.sigmoid(ii) * jnp.tanh(gg)
    h = jax.nn.sigmoid(oo) * jnp.tanh(c)
    h_ref[...] = h
    c_ref[...] = c


def _lstm_body_a(m1_ref, m2_ref, wih1_ref, whh1_ref, wih2_ref, whh2_ref,
                 bi1_ref, bi2_ref,
                 h1o_ref, c1o_ref, h2o_ref, c2o_ref, h1, c1, h2, c2):
    k = pl.program_id(1)

    @pl.when(k == 0)
    def _():
        h1[...] = jnp.zeros_like(h1)
        c1[...] = jnp.zeros_like(c1)
        h2[...] = jnp.zeros_like(h2)
        c2[...] = jnp.zeros_like(c2)

    _cell(m1_ref, h1, c1, wih1_ref, whh1_ref, bi1_ref)
    _cell(m2_ref, h2, c2, wih2_ref, whh2_ref, bi2_ref)

    @pl.when(k == KSEG - 1)
    def _():
        h1o_ref[...] = h1[...]
        c1o_ref[...] = c1[...]
        h2o_ref[...] = h2[...]
        c2o_ref[...] = c2[...]


def _lstm_body_m(m1_ref, m2_ref, h1i_ref, c1i_ref, h2i_ref, c2i_ref,
                 wih1_ref, whh1_ref, wih2_ref, whh2_ref,
                 bi1_ref, bi2_ref,
                 h1o_ref, c1o_ref, h2o_ref, c2o_ref, h1, c1, h2, c2):
    k = pl.program_id(1)

    @pl.when(k == 0)
    def _():
        h1[...] = h1i_ref[...]
        c1[...] = c1i_ref[...]
        h2[...] = h2i_ref[...]
        c2[...] = c2i_ref[...]

    _cell(m1_ref, h1, c1, wih1_ref, whh1_ref, bi1_ref)
    _cell(m2_ref, h2, c2, wih2_ref, whh2_ref, bi2_ref)

    @pl.when(k == KSEG - 1)
    def _():
        h1o_ref[...] = h1[...]
        c1o_ref[...] = c1[...]
        h2o_ref[...] = h2[...]
        c2o_ref[...] = c2[...]


def _lstm_body_b(m1_ref, m2_ref, h1i_ref, c1i_ref, h2i_ref, c2i_ref, x_ref,
                 wih1_ref, whh1_ref, wih2_ref, whh2_ref,
                 fcn1_ref, fcn2_ref, fcs_ref, bi1_ref, bi2_ref, boff_ref,
                 o_ref, h1, c1, h2, c2):
    k = pl.program_id(1)

    @pl.when(k == 0)
    def _():
        h1[...] = h1i_ref[...]
        c1[...] = c1i_ref[...]
        h2[...] = h2i_ref[...]
        c2[...] = c2i_ref[...]

    _cell(m1_ref, h1, c1, wih1_ref, whh1_ref, bi1_ref)
    _cell(m2_ref, h2, c2, wih2_ref, whh2_ref, bi2_ref)

    @pl.when(k == KSEG - 1)
    def _():
        o_ref[...] = (jnp.dot(x_ref[...], fcs_ref[...],
                              preferred_element_type=jnp.float32)
                      + jnp.dot(h1[...], fcn1_ref[...],
                                preferred_element_type=jnp.float32)
                      + jnp.dot(h2[...], fcn2_ref[...],
                                preferred_element_type=jnp.float32)
                      + boff_ref[...])


def _full(shape):
    return pl.BlockSpec(shape, lambda nb, k: tuple(0 for _ in shape))


_MAIL_SPEC = pl.BlockSpec((1, B, D), lambda nb, k: (k, nb, 0))
_ST_SPEC = pl.BlockSpec((B, D), lambda nb, k: (nb, 0))
_MAIL_SPEC_B = pl.BlockSpec((1, BB, D), lambda nb, k: (k, nb, 0))
_ST_SPEC_B = pl.BlockSpec((BB, D), lambda nb, k: (nb, 0))
_STATE = jax.ShapeDtypeStruct((N, D), jnp.float32)


def _tc_lstm_a(mail1, mail2, wih1T, whh1T, wih2T, whh2T, bi1, bi2):
    return pl.pallas_call(
        _lstm_body_a,
        grid=(P, KSEG),
        in_specs=[
            _MAIL_SPEC, _MAIL_SPEC,
            _full((D, 4 * D)), _full((D, 4 * D)),
            _full((D, 4 * D)), _full((D, 4 * D)),
            _full((1, 4 * D)), _full((1, 4 * D)),
        ],
        out_specs=[_ST_SPEC, _ST_SPEC, _ST_SPEC, _ST_SPEC],
        out_shape=[_STATE, _STATE, _STATE, _STATE],
        scratch_shapes=[pltpu.VMEM((B, D), jnp.float32)] * 4,
    )(mail1, mail2, wih1T, whh1T, wih2T, whh2T, bi1, bi2)


def _tc_lstm_m(mail1, mail2, st, wih1T, whh1T, wih2T, whh2T, bi1, bi2):
    return pl.pallas_call(
        _lstm_body_m,
        grid=(PB, KSEG),
        in_specs=[
            _MAIL_SPEC_B, _MAIL_SPEC_B,
            _ST_SPEC_B, _ST_SPEC_B, _ST_SPEC_B, _ST_SPEC_B,
            _full((D, 4 * D)), _full((D, 4 * D)),
            _full((D, 4 * D)), _full((D, 4 * D)),
            _full((1, 4 * D)), _full((1, 4 * D)),
        ],
        out_specs=[_ST_SPEC_B, _ST_SPEC_B, _ST_SPEC_B, _ST_SPEC_B],
        out_shape=[_STATE, _STATE, _STATE, _STATE],
        scratch_shapes=[pltpu.VMEM((BB, D), jnp.float32)] * 4,
    )(mail1, mail2, *st, wih1T, whh1T, wih2T, whh2T, bi1, bi2)


def _tc_lstm_b(mail1, mail2, st, x, wih1T, whh1T, wih2T, whh2T,
               fcn1T, fcn2T, fcsT, bi1, bi2, boff):
    return pl.pallas_call(
        _lstm_body_b,
        grid=(PB, KSEG),
        in_specs=[
            _MAIL_SPEC_B, _MAIL_SPEC_B,
            _ST_SPEC_B, _ST_SPEC_B, _ST_SPEC_B, _ST_SPEC_B,
            _ST_SPEC_B,
            _full((D, 4 * D)), _full((D, 4 * D)),
            _full((D, 4 * D)), _full((D, 4 * D)),
            _full((D, D)), _full((D, D)), _full((D, D)),
            _full((1, 4 * D)), _full((1, 4 * D)), _full((1, D)),
        ],
        out_specs=_ST_SPEC_B,
        out_shape=jax.ShapeDtypeStruct((N, D), jnp.float32),
        scratch_shapes=[pltpu.VMEM((BB, D), jnp.float32)] * 4,
    )(mail1, mail2, *st, x, wih1T, whh1T, wih2T, whh2T,
      fcn1T, fcn2T, fcsT, bi1, bi2, boff)


def kernel(x, edge_index, fc_self1, fc_neigh1, bias1,
           lstm1_Wih, lstm1_Whh, lstm1_bih, lstm1_bhh,
           fc_self2, fc_neigh2, bias2,
           lstm2_Wih, lstm2_Whh, lstm2_bih, lstm2_bhh):
    src = edge_index[0].astype(jnp.int32)
    seg_off = jnp.repeat(jnp.arange(DEG, dtype=jnp.int32) % KSEG * N, N)
    gidx = src.reshape(DEG, NCH, CH)
    sidx = (src + seg_off).reshape(DEG, NCH, CH)

    xmean = _tc_mean(x)

    wih1T = lstm1_Wih.T.astype(jnp.bfloat16)
    whh1T = lstm1_Whh.T.astype(jnp.bfloat16)
    wih2T = lstm2_Wih.T.astype(jnp.bfloat16)
    whh2T = lstm2_Whh.T.astype(jnp.bfloat16)
    bi1 = (lstm1_bih + lstm1_bhh).reshape(1, 4 * D)
    bi2 = (lstm2_bih + lstm2_bhh).reshape(1, 4 * D)
    fcn1T = fc_neigh1.T
    fcn2T = fc_neigh2.T
    fcsT = (fc_self1 + fc_self2).T
    boff = (bias1 + bias2).reshape(1, D) + xmean

    mails = [
        _sc_build_mailboxes(x, gidx[i * KSEG:(i + 1) * KSEG],
                            sidx[i * KSEG:(i + 1) * KSEG])
        for i in range(4)
    ]
    mails = [(a.reshape(KSEG, N, D), b.reshape(KSEG, N, D)) for a, b in mails]

    st = _tc_lstm_a(*mails[0], wih1T, whh1T, wih2T, whh2T, bi1, bi2)
    st = _tc_lstm_m(*mails[1], st, wih1T, whh1T, wih2T, whh2T, bi1, bi2)
    st = _tc_lstm_m(*mails[2], st, wih1T, whh1T, wih2T, whh2T, bi1, bi2)

    return _tc_lstm_b(*mails[3], st, x, wih1T, whh1T, wih2T, whh2T,
                      fcn1T, fcn2T, fcsT, bi1, bi2, boff)


# speedup vs baseline: 1.3151x; 1.0978x over previous
import functools

import jax
import jax.numpy as jnp
from jax import lax
from jax.experimental import pallas as pl
from jax.experimental.pallas import tpu as pltpu
from jax.experimental.pallas import tpu_sc as plsc

N = 10000
D = 128
DEG = 32
E = N * DEG

NC = 2
NS = 16
NW = NC * NS

CH = 80
NCH = N // CH

B = 5000
P = N // B
BB = 2000
PB = N // BB


D2 = D // 2

XLOAD_W = 10
XLOAD_R = N // XLOAD_W


KSEG = DEG // 4


def _sc_build_mailboxes(x, gidx, sidx):
    mesh = plsc.VectorSubcoreMesh(core_axis_name="c", subcore_axis_name="s")

    @functools.partial(
        pl.kernel,
        out_type=(
            jax.ShapeDtypeStruct((KSEG * N, D), jnp.float32),
            jax.ShapeDtypeStruct((KSEG * N, D), jnp.float32),
        ),
        mesh=mesh,
        scratch_types=[
            pltpu.VMEM_SHARED((N, D), jnp.float32),
            pltpu.VMEM((4, CH), jnp.int32),
            pltpu.VMEM((4, CH), jnp.int32),
            pltpu.VMEM((2, CH, D), jnp.float32),
            pltpu.VMEM((2, CH, D), jnp.float32),
            pltpu.SemaphoreType.DMA,
            pltpu.SemaphoreType.DMA,
            pltpu.SemaphoreType.DMA,
            pltpu.SemaphoreType.DMA((2,)),
            pltpu.SemaphoreType.DMA((2,)),
        ],
    )
    def sc_kernel(x_hbm, gidx_hbm, sidx_hbm, mail1_hbm, mail2_hbm,
                  x_sh, gi_c, si_c, buf1, buf2,
                  sem_i, sem_g1, sem_g2, sem_w1, sem_w2):
        s = lax.axis_index("s")
        w = s * NC + lax.axis_index("c")
        seg = w // 4
        q = w % 4
        base = seg * N
        lo = 31 * q + jnp.minimum(q, 1)
        hi = lo + 32 - jnp.minimum(q, 1)

        @pl.when(s < XLOAD_W)
        def _():
            pltpu.sync_copy(x_hbm.at[pl.ds(s * XLOAD_R, XLOAD_R)],
                            x_sh.at[pl.ds(s * XLOAD_R, XLOAD_R)])

        def fire_idx(i):
            b = lax.rem(i, 4)
            pltpu.async_copy(gidx_hbm.at[seg].at[i], gi_c.at[b], sem_i)
            pltpu.async_copy(sidx_hbm.at[seg].at[i], si_c.at[b], sem_i)

        def wait_idx(i):
            b = lax.rem(i, 4)
            pltpu.make_async_copy(gidx_hbm.at[0].at[0], gi_c.at[b], sem_i).wait()
            pltpu.make_async_copy(gidx_hbm.at[0].at[0], si_c.at[b], sem_i).wait()

        def wait_w(i):
            b = lax.rem(i, 2)
            pltpu.make_async_copy(
                buf1.at[0], mail1_hbm.at[pl.ds(base + i * CH, CH)],
                sem_w1.at[b]).wait()
            pltpu.make_async_copy(
                buf2.at[0], mail2_hbm.at[pl.ds(base + i * CH, CH)],
                sem_w2.at[b]).wait()

        fire_idx(lo)
        plsc.subcore_barrier()

        def body(i, carry):
            b = lax.rem(i, 2)

            @pl.when(i >= lo + 2)
            def _():
                wait_w(i - 2)

            wait_idx(i)

            @pl.when(i + 1 < hi)
            def _():
                fire_idx(i + 1)

            pltpu.async_copy(x_sh.at[gi_c.at[lax.rem(i, 4)]], buf1.at[b], sem_g1)
            pltpu.async_copy(x_sh.at[pl.ds(i * CH, CH)], buf2.at[b], sem_g2)
            pltpu.make_async_copy(x_hbm.at[pl.ds(0, CH)], buf1.at[b], sem_g1).wait()
            pltpu.make_async_copy(x_hbm.at[pl.ds(0, CH)], buf2.at[b], sem_g2).wait()

            pltpu.async_copy(buf1.at[b],
                             mail1_hbm.at[pl.ds(base + i * CH, CH)], sem_w1.at[b])
            pltpu.async_copy(buf2.at[b], mail2_hbm.at[si_c.at[lax.rem(i, 4)]],
                             sem_w2.at[b])
            return carry

        lax.fori_loop(lo, hi, body, 0)
        wait_w(hi - 2)
        wait_w(hi - 1)

    return sc_kernel(x, gidx, sidx)


def _mean_body(x_ref, o_ref):
    i = pl.program_id(0)

    @pl.when(i == 0)
    def _():
        o_ref[...] = jnp.zeros_like(o_ref)

    o_ref[...] += jnp.sum(x_ref[...], axis=0, keepdims=True) * (1.0 / N)


def _tc_mean(x):
    return pl.pallas_call(
        _mean_body,
        grid=(P,),
        in_specs=[pl.BlockSpec((B, D), lambda i: (i, 0))],
        out_specs=pl.BlockSpec((1, D), lambda i: (0, 0)),
        out_shape=jax.ShapeDtypeStruct((1, D), jnp.float32),
    )(x)


def _sig(z):
    return 0.5 * jnp.tanh(0.5 * z) + 0.5


def _cell(m_ref, h_ref, c_ref, wih_ref, whh_ref, bi_ref):
    hb = h_ref[...].astype(jnp.bfloat16)
    mb = m_ref[0].astype(jnp.bfloat16)
    gates = (jnp.dot(mb, wih_ref[...], preferred_element_type=jnp.float32)
             + jnp.dot(hb, whh_ref[...], preferred_element_type=jnp.float32)
             + bi_ref[...])
    ii = gates[:, 0:D]
    ff = gates[:, D:2 * D]
    gg = gates[:, 2 * D:3 * D]
    oo = gates[:, 3 * D:4 * D]
    c = _sig(ff) * c_ref[...] + _sig(ii) * jnp.tanh(gg)
    h = _sig(oo) * jnp.tanh(c)
    h_ref[...] = h
    c_ref[...] = c


def _lstm_body_a(m1_ref, m2_ref, wih1_ref, whh1_ref, wih2_ref, whh2_ref,
                 bi1_ref, bi2_ref,
                 h1o_ref, c1o_ref, h2o_ref, c2o_ref, h1, c1, h2, c2):
    k = pl.program_id(1)

    @pl.when(k == 0)
    def _():
        h1[...] = jnp.zeros_like(h1)
        c1[...] = jnp.zeros_like(c1)
        h2[...] = jnp.zeros_like(h2)
        c2[...] = jnp.zeros_like(c2)

    _cell(m1_ref, h1, c1, wih1_ref, whh1_ref, bi1_ref)
    _cell(m2_ref, h2, c2, wih2_ref, whh2_ref, bi2_ref)

    @pl.when(k == KSEG - 1)
    def _():
        h1o_ref[...] = h1[...]
        c1o_ref[...] = c1[...]
        h2o_ref[...] = h2[...]
        c2o_ref[...] = c2[...]


def _lstm_body_m(m1_ref, m2_ref, h1i_ref, c1i_ref, h2i_ref, c2i_ref,
                 wih1_ref, whh1_ref, wih2_ref, whh2_ref,
                 bi1_ref, bi2_ref,
                 h1o_ref, c1o_ref, h2o_ref, c2o_ref, h1, c1, h2, c2):
    k = pl.program_id(1)

    @pl.when(k == 0)
    def _():
        h1[...] = h1i_ref[...]
        c1[...] = c1i_ref[...]
        h2[...] = h2i_ref[...]
        c2[...] = c2i_ref[...]

    _cell(m1_ref, h1, c1, wih1_ref, whh1_ref, bi1_ref)
    _cell(m2_ref, h2, c2, wih2_ref, whh2_ref, bi2_ref)

    @pl.when(k == KSEG - 1)
    def _():
        h1o_ref[...] = h1[...]
        c1o_ref[...] = c1[...]
        h2o_ref[...] = h2[...]
        c2o_ref[...] = c2[...]


def _lstm_body_b(m1_ref, m2_ref, h1i_ref, c1i_ref, h2i_ref, c2i_ref, x_ref,
                 wih1_ref, whh1_ref, wih2_ref, whh2_ref,
                 fcn1_ref, fcn2_ref, fcs_ref, bi1_ref, bi2_ref, boff_ref,
                 o_ref, h1, c1, h2, c2):
    k = pl.program_id(1)

    @pl.when(k == 0)
    def _():
        h1[...] = h1i_ref[...]
        c1[...] = c1i_ref[...]
        h2[...] = h2i_ref[...]
        c2[...] = c2i_ref[...]

    _cell(m1_ref, h1, c1, wih1_ref, whh1_ref, bi1_ref)
    _cell(m2_ref, h2, c2, wih2_ref, whh2_ref, bi2_ref)

    @pl.when(k == KSEG - 1)
    def _():
        o_ref[...] = (jnp.dot(x_ref[...], fcs_ref[...],
                              preferred_element_type=jnp.float32)
                      + jnp.dot(h1[...], fcn1_ref[...],
                                preferred_element_type=jnp.float32)
                      + jnp.dot(h2[...], fcn2_ref[...],
                                preferred_element_type=jnp.float32)
                      + boff_ref[...])


def _full(shape):
    return pl.BlockSpec(shape, lambda nb, k: tuple(0 for _ in shape))


_MAIL_SPEC = pl.BlockSpec((1, B, D), lambda nb, k: (k, nb, 0))
_ST_SPEC = pl.BlockSpec((B, D), lambda nb, k: (nb, 0))
_MAIL_SPEC_B = pl.BlockSpec((1, BB, D), lambda nb, k: (k, nb, 0))
_ST_SPEC_B = pl.BlockSpec((BB, D), lambda nb, k: (nb, 0))
_STATE = jax.ShapeDtypeStruct((N, D), jnp.float32)


def _tc_lstm_a(mail1, mail2, wih1T, whh1T, wih2T, whh2T, bi1, bi2):
    return pl.pallas_call(
        _lstm_body_a,
        grid=(P, KSEG),
        in_specs=[
            _MAIL_SPEC, _MAIL_SPEC,
            _full((D, 4 * D)), _full((D, 4 * D)),
            _full((D, 4 * D)), _full((D, 4 * D)),
            _full((1, 4 * D)), _full((1, 4 * D)),
        ],
        out_specs=[_ST_SPEC, _ST_SPEC, _ST_SPEC, _ST_SPEC],
        out_shape=[_STATE, _STATE, _STATE, _STATE],
        scratch_shapes=[pltpu.VMEM((B, D), jnp.float32)] * 4,
    )(mail1, mail2, wih1T, whh1T, wih2T, whh2T, bi1, bi2)


def _tc_lstm_m(mail1, mail2, st, wih1T, whh1T, wih2T, whh2T, bi1, bi2):
    return pl.pallas_call(
        _lstm_body_m,
        grid=(PB, KSEG),
        in_specs=[
            _MAIL_SPEC_B, _MAIL_SPEC_B,
            _ST_SPEC_B, _ST_SPEC_B, _ST_SPEC_B, _ST_SPEC_B,
            _full((D, 4 * D)), _full((D, 4 * D)),
            _full((D, 4 * D)), _full((D, 4 * D)),
            _full((1, 4 * D)), _full((1, 4 * D)),
        ],
        out_specs=[_ST_SPEC_B, _ST_SPEC_B, _ST_SPEC_B, _ST_SPEC_B],
        out_shape=[_STATE, _STATE, _STATE, _STATE],
        scratch_shapes=[pltpu.VMEM((BB, D), jnp.float32)] * 4,
    )(mail1, mail2, *st, wih1T, whh1T, wih2T, whh2T, bi1, bi2)


def _tc_lstm_b(mail1, mail2, st, x, wih1T, whh1T, wih2T, whh2T,
               fcn1T, fcn2T, fcsT, bi1, bi2, boff):
    return pl.pallas_call(
        _lstm_body_b,
        grid=(PB, KSEG),
        in_specs=[
            _MAIL_SPEC_B, _MAIL_SPEC_B,
            _ST_SPEC_B, _ST_SPEC_B, _ST_SPEC_B, _ST_SPEC_B,
            _ST_SPEC_B,
            _full((D, 4 * D)), _full((D, 4 * D)),
            _full((D, 4 * D)), _full((D, 4 * D)),
            _full((D, D)), _full((D, D)), _full((D, D)),
            _full((1, 4 * D)), _full((1, 4 * D)), _full((1, D)),
        ],
        out_specs=_ST_SPEC_B,
        out_shape=jax.ShapeDtypeStruct((N, D), jnp.float32),
        scratch_shapes=[pltpu.VMEM((BB, D), jnp.float32)] * 4,
    )(mail1, mail2, *st, x, wih1T, whh1T, wih2T, whh2T,
      fcn1T, fcn2T, fcsT, bi1, bi2, boff)


def kernel(x, edge_index, fc_self1, fc_neigh1, bias1,
           lstm1_Wih, lstm1_Whh, lstm1_bih, lstm1_bhh,
           fc_self2, fc_neigh2, bias2,
           lstm2_Wih, lstm2_Whh, lstm2_bih, lstm2_bhh):
    src = edge_index[0].astype(jnp.int32)
    seg_off = jnp.repeat(jnp.arange(DEG, dtype=jnp.int32) % KSEG * N, N)
    gidx = src.reshape(DEG, NCH, CH)
    sidx = (src + seg_off).reshape(DEG, NCH, CH)

    xmean = _tc_mean(x)

    wih1T = lstm1_Wih.T.astype(jnp.bfloat16)
    whh1T = lstm1_Whh.T.astype(jnp.bfloat16)
    wih2T = lstm2_Wih.T.astype(jnp.bfloat16)
    whh2T = lstm2_Whh.T.astype(jnp.bfloat16)
    bi1 = (lstm1_bih + lstm1_bhh).reshape(1, 4 * D)
    bi2 = (lstm2_bih + lstm2_bhh).reshape(1, 4 * D)
    fcn1T = fc_neigh1.T
    fcn2T = fc_neigh2.T
    fcsT = (fc_self1 + fc_self2).T
    boff = (bias1 + bias2).reshape(1, D) + xmean

    mails = [
        _sc_build_mailboxes(x, gidx[i * KSEG:(i + 1) * KSEG],
                            sidx[i * KSEG:(i + 1) * KSEG])
        for i in range(4)
    ]
    mails = [(a.reshape(KSEG, N, D), b.reshape(KSEG, N, D)) for a, b in mails]

    st = _tc_lstm_a(*mails[0], wih1T, whh1T, wih2T, whh2T, bi1, bi2)
    st = _tc_lstm_m(*mails[1], st, wih1T, whh1T, wih2T, whh2T, bi1, bi2)
    st = _tc_lstm_m(*mails[2], st, wih1T, whh1T, wih2T, whh2T, bi1, bi2)

    return _tc_lstm_b(*mails[3], st, x, wih1T, whh1T, wih2T, whh2T,
                      fcn1T, fcn2T, fcsT, bi1, bi2, boff)


# speedup vs baseline: 1.7498x; 1.3305x over previous
import functools

import jax
import jax.numpy as jnp
from jax import lax
from jax.experimental import pallas as pl
from jax.experimental.pallas import tpu as pltpu
from jax.experimental.pallas import tpu_sc as plsc

N = 10000
D = 128
DEG = 32
E = N * DEG

NC = 2
NS = 16
NW = NC * NS

CH = 80
NCH = N // CH

B = 5000
P = N // B
BB = 2000
PB = N // BB


D2 = D // 2

XLOAD_W = 10
XLOAD_R = N // XLOAD_W


KSEG = DEG // 4


def _sc_build_mailboxes(x, gidx, sidx):
    mesh = plsc.VectorSubcoreMesh(core_axis_name="c", subcore_axis_name="s")

    @functools.partial(
        pl.kernel,
        out_type=(
            jax.ShapeDtypeStruct((KSEG * N, D), jnp.float32),
            jax.ShapeDtypeStruct((KSEG * N, D), jnp.float32),
        ),
        mesh=mesh,
        scratch_types=[
            pltpu.VMEM_SHARED((N, D), jnp.float32),
            pltpu.VMEM((4, CH), jnp.int32),
            pltpu.VMEM((4, CH), jnp.int32),
            pltpu.VMEM((2, CH, D), jnp.float32),
            pltpu.VMEM((2, CH, D), jnp.float32),
            pltpu.SemaphoreType.DMA,
            pltpu.SemaphoreType.DMA,
            pltpu.SemaphoreType.DMA,
            pltpu.SemaphoreType.DMA((2,)),
            pltpu.SemaphoreType.DMA((2,)),
        ],
    )
    def sc_kernel(x_hbm, gidx_hbm, sidx_hbm, mail1_hbm, mail2_hbm,
                  x_sh, gi_c, si_c, buf1, buf2,
                  sem_i, sem_g1, sem_g2, sem_w1, sem_w2):
        s = lax.axis_index("s")
        w = s * NC + lax.axis_index("c")
        seg = w // 4
        q = w % 4
        base = seg * N
        lo = 31 * q + jnp.minimum(q, 1)
        hi = lo + 32 - jnp.minimum(q, 1)

        @pl.when(s < XLOAD_W)
        def _():
            pltpu.sync_copy(x_hbm.at[pl.ds(s * XLOAD_R, XLOAD_R)],
                            x_sh.at[pl.ds(s * XLOAD_R, XLOAD_R)])

        def fire_idx(i):
            b = lax.rem(i, 4)
            pltpu.async_copy(gidx_hbm.at[seg].at[i], gi_c.at[b], sem_i)
            pltpu.async_copy(sidx_hbm.at[seg].at[i], si_c.at[b], sem_i)

        def wait_idx(i):
            b = lax.rem(i, 4)
            pltpu.make_async_copy(gidx_hbm.at[0].at[0], gi_c.at[b], sem_i).wait()
            pltpu.make_async_copy(gidx_hbm.at[0].at[0], si_c.at[b], sem_i).wait()

        def wait_w(i):
            b = lax.rem(i, 2)
            pltpu.make_async_copy(
                buf1.at[0], mail1_hbm.at[pl.ds(base + i * CH, CH)],
                sem_w1.at[b]).wait()
            pltpu.make_async_copy(
                buf2.at[0], mail2_hbm.at[pl.ds(base + i * CH, CH)],
                sem_w2.at[b]).wait()

        fire_idx(lo)
        plsc.subcore_barrier()

        def body(i, carry):
            b = lax.rem(i, 2)

            @pl.when(i >= lo + 2)
            def _():
                wait_w(i - 2)

            wait_idx(i)

            @pl.when(i + 1 < hi)
            def _():
                fire_idx(i + 1)

            pltpu.async_copy(x_sh.at[gi_c.at[lax.rem(i, 4)]], buf1.at[b], sem_g1)
            pltpu.async_copy(x_sh.at[pl.ds(i * CH, CH)], buf2.at[b], sem_g2)
            pltpu.make_async_copy(x_hbm.at[pl.ds(0, CH)], buf1.at[b], sem_g1).wait()
            pltpu.make_async_copy(x_hbm.at[pl.ds(0, CH)], buf2.at[b], sem_g2).wait()

            pltpu.async_copy(buf1.at[b],
                             mail1_hbm.at[pl.ds(base + i * CH, CH)], sem_w1.at[b])
            pltpu.async_copy(buf2.at[b], mail2_hbm.at[si_c.at[lax.rem(i, 4)]],
                             sem_w2.at[b])
            return carry

        lax.fori_loop(lo, hi, body, 0)
        wait_w(hi - 2)
        wait_w(hi - 1)

    return sc_kernel(x, gidx, sidx)


def _mean_body(x_ref, o_ref):
    i = pl.program_id(0)

    @pl.when(i == 0)
    def _():
        o_ref[...] = jnp.zeros_like(o_ref)

    o_ref[...] += jnp.sum(x_ref[...], axis=0, keepdims=True) * (1.0 / N)


def _tc_mean(x):
    return pl.pallas_call(
        _mean_body,
        grid=(P,),
        in_specs=[pl.BlockSpec((B, D), lambda i: (i, 0))],
        out_specs=pl.BlockSpec((1, D), lambda i: (0, 0)),
        out_shape=jax.ShapeDtypeStruct((1, D), jnp.float32),
    )(x)


def _sig(z):
    return 0.5 * jnp.tanh(0.5 * z) + 0.5


def _cell(m_ref, h_ref, c_ref, w_ref, bi_ref):
    mh = jnp.concatenate([m_ref[0].astype(jnp.bfloat16),
                          h_ref[...].astype(jnp.bfloat16)], axis=1)
    gates = (jnp.dot(mh, w_ref[...], preferred_element_type=jnp.float32)
             + bi_ref[...])
    ii = gates[:, 0:D]
    ff = gates[:, D:2 * D]
    gg = gates[:, 2 * D:3 * D]
    oo = gates[:, 3 * D:4 * D]
    c = _sig(ff) * c_ref[...] + _sig(ii) * jnp.tanh(gg)
    h = _sig(oo) * jnp.tanh(c)
    h_ref[...] = h
    c_ref[...] = c


def _lstm_body_a(m1_ref, m2_ref, wih1_ref, wih2_ref,
                 bi1_ref, bi2_ref,
                 h1o_ref, c1o_ref, h2o_ref, c2o_ref, h1, c1, h2, c2):
    k = pl.program_id(1)

    @pl.when(k == 0)
    def _():
        h1[...] = jnp.zeros_like(h1)
        c1[...] = jnp.zeros_like(c1)
        h2[...] = jnp.zeros_like(h2)
        c2[...] = jnp.zeros_like(c2)

    _cell(m1_ref, h1, c1, wih1_ref, bi1_ref)
    _cell(m2_ref, h2, c2, wih2_ref, bi2_ref)

    @pl.when(k == KSEG - 1)
    def _():
        h1o_ref[...] = h1[...]
        c1o_ref[...] = c1[...]
        h2o_ref[...] = h2[...]
        c2o_ref[...] = c2[...]


def _lstm_body_m(m1_ref, m2_ref, h1i_ref, c1i_ref, h2i_ref, c2i_ref,
                 wih1_ref, wih2_ref,
                 bi1_ref, bi2_ref,
                 h1o_ref, c1o_ref, h2o_ref, c2o_ref, h1, c1, h2, c2):
    k = pl.program_id(1)

    @pl.when(k == 0)
    def _():
        h1[...] = h1i_ref[...]
        c1[...] = c1i_ref[...]
        h2[...] = h2i_ref[...]
        c2[...] = c2i_ref[...]

    _cell(m1_ref, h1, c1, wih1_ref, bi1_ref)
    _cell(m2_ref, h2, c2, wih2_ref, bi2_ref)

    @pl.when(k == KSEG - 1)
    def _():
        h1o_ref[...] = h1[...]
        c1o_ref[...] = c1[...]
        h2o_ref[...] = h2[...]
        c2o_ref[...] = c2[...]


def _lstm_body_b(m1_ref, m2_ref, h1i_ref, c1i_ref, h2i_ref, c2i_ref, x_ref,
                 wih1_ref, wih2_ref,
                 fcn1_ref, fcn2_ref, fcs_ref, bi1_ref, bi2_ref, boff_ref,
                 o_ref, h1, c1, h2, c2):
    k = pl.program_id(1)

    @pl.when(k == 0)
    def _():
        h1[...] = h1i_ref[...]
        c1[...] = c1i_ref[...]
        h2[...] = h2i_ref[...]
        c2[...] = c2i_ref[...]

    _cell(m1_ref, h1, c1, wih1_ref, bi1_ref)
    _cell(m2_ref, h2, c2, wih2_ref, bi2_ref)

    @pl.when(k == KSEG - 1)
    def _():
        o_ref[...] = (jnp.dot(x_ref[...], fcs_ref[...],
                              preferred_element_type=jnp.float32)
                      + jnp.dot(h1[...], fcn1_ref[...],
                                preferred_element_type=jnp.float32)
                      + jnp.dot(h2[...], fcn2_ref[...],
                                preferred_element_type=jnp.float32)
                      + boff_ref[...])


def _full(shape):
    return pl.BlockSpec(shape, lambda nb, k: tuple(0 for _ in shape))


_MAIL_SPEC = pl.BlockSpec((1, B, D), lambda nb, k: (k, nb, 0))
_ST_SPEC = pl.BlockSpec((B, D), lambda nb, k: (nb, 0))
_MAIL_SPEC_B = pl.BlockSpec((1, BB, D), lambda nb, k: (k, nb, 0))
_ST_SPEC_B = pl.BlockSpec((BB, D), lambda nb, k: (nb, 0))
_STATE = jax.ShapeDtypeStruct((N, D), jnp.float32)


def _tc_lstm_a(mail1, mail2, w1, w2, bi1, bi2):
    return pl.pallas_call(
        _lstm_body_a,
        grid=(P, KSEG),
        in_specs=[
            _MAIL_SPEC, _MAIL_SPEC,
            _full((2 * D, 4 * D)), _full((2 * D, 4 * D)),
            _full((1, 4 * D)), _full((1, 4 * D)),
        ],
        out_specs=[_ST_SPEC, _ST_SPEC, _ST_SPEC, _ST_SPEC],
        out_shape=[_STATE, _STATE, _STATE, _STATE],
        scratch_shapes=[pltpu.VMEM((B, D), jnp.float32)] * 4,
    )(mail1, mail2, w1, w2, bi1, bi2)


def _tc_lstm_m(mail1, mail2, st, w1, w2, bi1, bi2):
    return pl.pallas_call(
        _lstm_body_m,
        grid=(PB, KSEG),
        in_specs=[
            _MAIL_SPEC_B, _MAIL_SPEC_B,
            _ST_SPEC_B, _ST_SPEC_B, _ST_SPEC_B, _ST_SPEC_B,
            _full((2 * D, 4 * D)), _full((2 * D, 4 * D)),
            _full((1, 4 * D)), _full((1, 4 * D)),
        ],
        out_specs=[_ST_SPEC_B, _ST_SPEC_B, _ST_SPEC_B, _ST_SPEC_B],
        out_shape=[_STATE, _STATE, _STATE, _STATE],
        scratch_shapes=[pltpu.VMEM((BB, D), jnp.float32)] * 4,
    )(mail1, mail2, *st, w1, w2, bi1, bi2)


def _tc_lstm_b(mail1, mail2, st, x, w1, w2,
               fcn1T, fcn2T, fcsT, bi1, bi2, boff):
    return pl.pallas_call(
        _lstm_body_b,
        grid=(PB, KSEG),
        in_specs=[
            _MAIL_SPEC_B, _MAIL_SPEC_B,
            _ST_SPEC_B, _ST_SPEC_B, _ST_SPEC_B, _ST_SPEC_B,
            _ST_SPEC_B,
            _full((2 * D, 4 * D)), _full((2 * D, 4 * D)),
            _full((D, D)), _full((D, D)), _full((D, D)),
            _full((1, 4 * D)), _full((1, 4 * D)), _full((1, D)),
        ],
        out_specs=_ST_SPEC_B,
        out_shape=jax.ShapeDtypeStruct((N, D), jnp.float32),
        scratch_shapes=[pltpu.VMEM((BB, D), jnp.float32)] * 4,
    )(mail1, mail2, *st, x, w1, w2,
      fcn1T, fcn2T, fcsT, bi1, bi2, boff)


def kernel(x, edge_index, fc_self1, fc_neigh1, bias1,
           lstm1_Wih, lstm1_Whh, lstm1_bih, lstm1_bhh,
           fc_self2, fc_neigh2, bias2,
           lstm2_Wih, lstm2_Whh, lstm2_bih, lstm2_bhh):
    src = edge_index[0].astype(jnp.int32)
    seg_off = jnp.repeat(jnp.arange(DEG, dtype=jnp.int32) % KSEG * N, N)
    gidx = src.reshape(DEG, NCH, CH)
    sidx = (src + seg_off).reshape(DEG, NCH, CH)

    xmean = _tc_mean(x)

    w1 = jnp.concatenate([lstm1_Wih.T, lstm1_Whh.T],
                         axis=0).astype(jnp.bfloat16)
    w2 = jnp.concatenate([lstm2_Wih.T, lstm2_Whh.T],
                         axis=0).astype(jnp.bfloat16)
    bi1 = (lstm1_bih + lstm1_bhh).reshape(1, 4 * D)
    bi2 = (lstm2_bih + lstm2_bhh).reshape(1, 4 * D)
    fcn1T = fc_neigh1.T
    fcn2T = fc_neigh2.T
    fcsT = (fc_self1 + fc_self2).T
    boff = (bias1 + bias2).reshape(1, D) + xmean

    mails = [
        _sc_build_mailboxes(x, gidx[i * KSEG:(i + 1) * KSEG],
                            sidx[i * KSEG:(i + 1) * KSEG])
        for i in range(4)
    ]
    mails = [(a.reshape(KSEG, N, D), b.reshape(KSEG, N, D)) for a, b in mails]

    st = _tc_lstm_a(*mails[0], w1, w2, bi1, bi2)
    st = _tc_lstm_m(*mails[1], st, w1, w2, bi1, bi2)
    st = _tc_lstm_m(*mails[2], st, w1, w2, bi1, bi2)

    return _tc_lstm_b(*mails[3], st, x, w1, w2,
                      fcn1T, fcn2T, fcsT, bi1, bi2, boff)


# speedup vs baseline: 1.8201x; 1.0402x over previous
import functools

import jax
import jax.numpy as jnp
from jax import lax
from jax.experimental import pallas as pl
from jax.experimental.pallas import tpu as pltpu
from jax.experimental.pallas import tpu_sc as plsc

N = 10000
D = 128
DEG = 32
E = N * DEG

NC = 2
NS = 16
NW = NC * NS

CH = 80
NCH = N // CH

B = 5000
P = N // B
BB = 2000
PB = N // BB


D2 = D // 2

XLOAD_W = 10
XLOAD_R = N // XLOAD_W


KSEG = DEG // 4


def _sc_build_mailboxes(x, gidx, sidx):
    mesh = plsc.VectorSubcoreMesh(core_axis_name="c", subcore_axis_name="s")

    @functools.partial(
        pl.kernel,
        out_type=(
            jax.ShapeDtypeStruct((KSEG * N, D), jnp.float32),
            jax.ShapeDtypeStruct((KSEG * N, D), jnp.float32),
        ),
        mesh=mesh,
        scratch_types=[
            pltpu.VMEM_SHARED((N, D), jnp.float32),
            pltpu.VMEM((4, CH), jnp.int32),
            pltpu.VMEM((4, CH), jnp.int32),
            pltpu.VMEM((2, CH, D), jnp.float32),
            pltpu.VMEM((2, CH, D), jnp.float32),
            pltpu.SemaphoreType.DMA,
            pltpu.SemaphoreType.DMA,
            pltpu.SemaphoreType.DMA,
            pltpu.SemaphoreType.DMA((2,)),
            pltpu.SemaphoreType.DMA((2,)),
        ],
    )
    def sc_kernel(x_hbm, gidx_hbm, sidx_hbm, mail1_hbm, mail2_hbm,
                  x_sh, gi_c, si_c, buf1, buf2,
                  sem_i, sem_g1, sem_g2, sem_w1, sem_w2):
        s = lax.axis_index("s")
        w = s * NC + lax.axis_index("c")
        seg = w // 4
        q = w % 4
        base = seg * N
        lo = 31 * q + jnp.minimum(q, 1)
        hi = lo + 32 - jnp.minimum(q, 1)

        @pl.when(s < XLOAD_W)
        def _():
            pltpu.sync_copy(x_hbm.at[pl.ds(s * XLOAD_R, XLOAD_R)],
                            x_sh.at[pl.ds(s * XLOAD_R, XLOAD_R)])

        def fire_idx(i):
            b = lax.rem(i, 4)
            pltpu.async_copy(gidx_hbm.at[seg].at[i], gi_c.at[b], sem_i)
            pltpu.async_copy(sidx_hbm.at[seg].at[i], si_c.at[b], sem_i)

        def wait_idx(i):
            b = lax.rem(i, 4)
            pltpu.make_async_copy(gidx_hbm.at[0].at[0], gi_c.at[b], sem_i).wait()
            pltpu.make_async_copy(gidx_hbm.at[0].at[0], si_c.at[b], sem_i).wait()

        def wait_w(i):
            b = lax.rem(i, 2)
            pltpu.make_async_copy(
                buf1.at[0], mail1_hbm.at[pl.ds(base + i * CH, CH)],
                sem_w1.at[b]).wait()
            pltpu.make_async_copy(
                buf2.at[0], mail2_hbm.at[pl.ds(base + i * CH, CH)],
                sem_w2.at[b]).wait()

        fire_idx(lo)
        plsc.subcore_barrier()

        def body(i, carry):
            b = lax.rem(i, 2)

            @pl.when(i >= lo + 2)
            def _():
                wait_w(i - 2)

            wait_idx(i)

            @pl.when(i + 1 < hi)
            def _():
                fire_idx(i + 1)

            pltpu.async_copy(x_sh.at[gi_c.at[lax.rem(i, 4)]], buf1.at[b], sem_g1)
            pltpu.async_copy(x_sh.at[pl.ds(i * CH, CH)], buf2.at[b], sem_g2)
            pltpu.make_async_copy(x_hbm.at[pl.ds(0, CH)], buf1.at[b], sem_g1).wait()
            pltpu.make_async_copy(x_hbm.at[pl.ds(0, CH)], buf2.at[b], sem_g2).wait()

            pltpu.async_copy(buf1.at[b],
                             mail1_hbm.at[pl.ds(base + i * CH, CH)], sem_w1.at[b])
            pltpu.async_copy(buf2.at[b], mail2_hbm.at[si_c.at[lax.rem(i, 4)]],
                             sem_w2.at[b])
            return carry

        lax.fori_loop(lo, hi, body, 0)
        wait_w(hi - 2)
        wait_w(hi - 1)

    return sc_kernel(x, gidx, sidx)


def _mean_body(x_ref, o_ref):
    i = pl.program_id(0)

    @pl.when(i == 0)
    def _():
        o_ref[...] = jnp.zeros_like(o_ref)

    o_ref[...] += jnp.sum(x_ref[...], axis=0, keepdims=True) * (1.0 / N)


def _tc_mean(x):
    return pl.pallas_call(
        _mean_body,
        grid=(P,),
        in_specs=[pl.BlockSpec((B, D), lambda i: (i, 0))],
        out_specs=pl.BlockSpec((1, D), lambda i: (0, 0)),
        out_shape=jax.ShapeDtypeStruct((1, D), jnp.float32),
    )(x)


def _sig(z):
    return 0.5 * jnp.tanh(0.5 * z) + 0.5


def _cell(m_ref, h_ref, c_ref, w_ref, bi_ref):
    mh = jnp.concatenate([m_ref[0].astype(jnp.bfloat16),
                          h_ref[...].astype(jnp.bfloat16)], axis=1)
    gates = (jnp.dot(mh, w_ref[...], preferred_element_type=jnp.float32)
             + bi_ref[...])
    ii = gates[:, 0:D]
    ff = gates[:, D:2 * D]
    gg = gates[:, 2 * D:3 * D]
    oo = gates[:, 3 * D:4 * D]
    c = _sig(ff) * c_ref[...] + _sig(ii) * jnp.tanh(gg)
    h = _sig(oo) * jnp.tanh(c)
    h_ref[...] = h
    c_ref[...] = c


def _lstm_body_a(m1_ref, m2_ref, wih1_ref, wih2_ref,
                 bi1_ref, bi2_ref,
                 h1o_ref, c1o_ref, h2o_ref, c2o_ref, h1, c1, h2, c2):
    k = pl.program_id(1)

    @pl.when(k == 0)
    def _():
        h1[...] = jnp.zeros_like(h1)
        c1[...] = jnp.zeros_like(c1)
        h2[...] = jnp.zeros_like(h2)
        c2[...] = jnp.zeros_like(c2)

    _cell(m1_ref, h1, c1, wih1_ref, bi1_ref)
    _cell(m2_ref, h2, c2, wih2_ref, bi2_ref)

    @pl.when(k == KSEG - 1)
    def _():
        h1o_ref[...] = h1[...].astype(jnp.bfloat16)
        c1o_ref[...] = c1[...].astype(jnp.bfloat16)
        h2o_ref[...] = h2[...].astype(jnp.bfloat16)
        c2o_ref[...] = c2[...].astype(jnp.bfloat16)


def _lstm_body_m(m1_ref, m2_ref, h1i_ref, c1i_ref, h2i_ref, c2i_ref,
                 wih1_ref, wih2_ref,
                 bi1_ref, bi2_ref,
                 h1o_ref, c1o_ref, h2o_ref, c2o_ref, h1, c1, h2, c2):
    k = pl.program_id(1)

    @pl.when(k == 0)
    def _():
        h1[...] = h1i_ref[...].astype(jnp.float32)
        c1[...] = c1i_ref[...].astype(jnp.float32)
        h2[...] = h2i_ref[...].astype(jnp.float32)
        c2[...] = c2i_ref[...].astype(jnp.float32)

    _cell(m1_ref, h1, c1, wih1_ref, bi1_ref)
    _cell(m2_ref, h2, c2, wih2_ref, bi2_ref)

    @pl.when(k == KSEG - 1)
    def _():
        h1o_ref[...] = h1[...].astype(jnp.bfloat16)
        c1o_ref[...] = c1[...].astype(jnp.bfloat16)
        h2o_ref[...] = h2[...].astype(jnp.bfloat16)
        c2o_ref[...] = c2[...].astype(jnp.bfloat16)


def _lstm_body_b(m1_ref, m2_ref, h1i_ref, c1i_ref, h2i_ref, c2i_ref, x_ref,
                 wih1_ref, wih2_ref,
                 fcn1_ref, fcn2_ref, fcs_ref, bi1_ref, bi2_ref, boff_ref,
                 o_ref, h1, c1, h2, c2):
    k = pl.program_id(1)

    @pl.when(k == 0)
    def _():
        h1[...] = h1i_ref[...].astype(jnp.float32)
        c1[...] = c1i_ref[...].astype(jnp.float32)
        h2[...] = h2i_ref[...].astype(jnp.float32)
        c2[...] = c2i_ref[...].astype(jnp.float32)

    _cell(m1_ref, h1, c1, wih1_ref, bi1_ref)
    _cell(m2_ref, h2, c2, wih2_ref, bi2_ref)

    @pl.when(k == KSEG - 1)
    def _():
        o_ref[...] = (jnp.dot(x_ref[...], fcs_ref[...],
                              preferred_element_type=jnp.float32)
                      + jnp.dot(h1[...], fcn1_ref[...],
                                preferred_element_type=jnp.float32)
                      + jnp.dot(h2[...], fcn2_ref[...],
                                preferred_element_type=jnp.float32)
                      + boff_ref[...])


def _full(shape):
    return pl.BlockSpec(shape, lambda nb, k: tuple(0 for _ in shape))


_MAIL_SPEC = pl.BlockSpec((1, B, D), lambda nb, k: (k, nb, 0))
_ST_SPEC = pl.BlockSpec((B, D), lambda nb, k: (nb, 0))
_MAIL_SPEC_B = pl.BlockSpec((1, BB, D), lambda nb, k: (k, nb, 0))
_ST_SPEC_B = pl.BlockSpec((BB, D), lambda nb, k: (nb, 0))
_STATE = jax.ShapeDtypeStruct((N, D), jnp.bfloat16)


def _tc_lstm_a(mail1, mail2, w1, w2, bi1, bi2):
    return pl.pallas_call(
        _lstm_body_a,
        grid=(P, KSEG),
        in_specs=[
            _MAIL_SPEC, _MAIL_SPEC,
            _full((2 * D, 4 * D)), _full((2 * D, 4 * D)),
            _full((1, 4 * D)), _full((1, 4 * D)),
        ],
        out_specs=[_ST_SPEC, _ST_SPEC, _ST_SPEC, _ST_SPEC],
        out_shape=[_STATE, _STATE, _STATE, _STATE],
        scratch_shapes=[pltpu.VMEM((B, D), jnp.float32)] * 4,
    )(mail1, mail2, w1, w2, bi1, bi2)


def _tc_lstm_m(mail1, mail2, st, w1, w2, bi1, bi2):
    return pl.pallas_call(
        _lstm_body_m,
        grid=(PB, KSEG),
        in_specs=[
            _MAIL_SPEC_B, _MAIL_SPEC_B,
            _ST_SPEC_B, _ST_SPEC_B, _ST_SPEC_B, _ST_SPEC_B,
            _full((2 * D, 4 * D)), _full((2 * D, 4 * D)),
            _full((1, 4 * D)), _full((1, 4 * D)),
        ],
        out_specs=[_ST_SPEC_B, _ST_SPEC_B, _ST_SPEC_B, _ST_SPEC_B],
        out_shape=[_STATE, _STATE, _STATE, _STATE],
        scratch_shapes=[pltpu.VMEM((BB, D), jnp.float32)] * 4,
    )(mail1, mail2, *st, w1, w2, bi1, bi2)


def _tc_lstm_b(mail1, mail2, st, x, w1, w2,
               fcn1T, fcn2T, fcsT, bi1, bi2, boff):
    return pl.pallas_call(
        _lstm_body_b,
        grid=(PB, KSEG),
        in_specs=[
            _MAIL_SPEC_B, _MAIL_SPEC_B,
            _ST_SPEC_B, _ST_SPEC_B, _ST_SPEC_B, _ST_SPEC_B,
            _ST_SPEC_B,
            _full((2 * D, 4 * D)), _full((2 * D, 4 * D)),
            _full((D, D)), _full((D, D)), _full((D, D)),
            _full((1, 4 * D)), _full((1, 4 * D)), _full((1, D)),
        ],
        out_specs=_ST_SPEC_B,
        out_shape=jax.ShapeDtypeStruct((N, D), jnp.float32),
        scratch_shapes=[pltpu.VMEM((BB, D), jnp.float32)] * 4,
    )(mail1, mail2, *st, x, w1, w2,
      fcn1T, fcn2T, fcsT, bi1, bi2, boff)


def kernel(x, edge_index, fc_self1, fc_neigh1, bias1,
           lstm1_Wih, lstm1_Whh, lstm1_bih, lstm1_bhh,
           fc_self2, fc_neigh2, bias2,
           lstm2_Wih, lstm2_Whh, lstm2_bih, lstm2_bhh):
    src = edge_index[0].astype(jnp.int32)
    seg_off = jnp.repeat(jnp.arange(DEG, dtype=jnp.int32) % KSEG * N, N)
    gidx = src.reshape(DEG, NCH, CH)
    sidx = (src + seg_off).reshape(DEG, NCH, CH)

    xmean = _tc_mean(x)

    w1 = jnp.concatenate([lstm1_Wih.T, lstm1_Whh.T],
                         axis=0).astype(jnp.bfloat16)
    w2 = jnp.concatenate([lstm2_Wih.T, lstm2_Whh.T],
                         axis=0).astype(jnp.bfloat16)
    bi1 = (lstm1_bih + lstm1_bhh).reshape(1, 4 * D)
    bi2 = (lstm2_bih + lstm2_bhh).reshape(1, 4 * D)
    fcn1T = fc_neigh1.T
    fcn2T = fc_neigh2.T
    fcsT = (fc_self1 + fc_self2).T
    boff = (bias1 + bias2).reshape(1, D) + xmean

    mails = [
        _sc_build_mailboxes(x, gidx[i * KSEG:(i + 1) * KSEG],
                            sidx[i * KSEG:(i + 1) * KSEG])
        for i in range(4)
    ]
    mails = [(a.reshape(KSEG, N, D), b.reshape(KSEG, N, D)) for a, b in mails]

    st = _tc_lstm_a(*mails[0], w1, w2, bi1, bi2)
    st = _tc_lstm_m(*mails[1], st, w1, w2, bi1, bi2)
    st = _tc_lstm_m(*mails[2], st, w1, w2, bi1, bi2)

    return _tc_lstm_b(*mails[3], st, x, w1, w2,
                      fcn1T, fcn2T, fcsT, bi1, bi2, boff)


# speedup vs baseline: 1.8201x; 1.0000x over previous
import functools

import jax
import jax.numpy as jnp
from jax import lax
from jax.experimental import pallas as pl
from jax.experimental.pallas import tpu as pltpu
from jax.experimental.pallas import tpu_sc as plsc

N = 10000
D = 128
DEG = 32
E = N * DEG

NC = 2
NS = 16
NW = NC * NS

CH = 80
NCH = N // CH

B = 5000
P = N // B
BB = 2000
PB = N // BB


XLOAD_W = 10
XLOAD_R = N // XLOAD_W


KSEG = DEG // 4


def _sc_build_mailboxes(x, gidx, sidx):
    mesh = plsc.VectorSubcoreMesh(core_axis_name="c", subcore_axis_name="s")

    @functools.partial(
        pl.kernel,
        out_type=(
            jax.ShapeDtypeStruct((KSEG * N, D), jnp.float32),
            jax.ShapeDtypeStruct((KSEG * N, D), jnp.float32),
        ),
        mesh=mesh,
        scratch_types=[
            pltpu.VMEM_SHARED((N, D), jnp.float32),
            pltpu.VMEM((4, CH), jnp.int32),
            pltpu.VMEM((4, CH), jnp.int32),
            pltpu.VMEM((2, CH, D), jnp.float32),
            pltpu.VMEM((2, CH, D), jnp.float32),
            pltpu.SemaphoreType.DMA,
            pltpu.SemaphoreType.DMA,
            pltpu.SemaphoreType.DMA,
            pltpu.SemaphoreType.DMA((2,)),
            pltpu.SemaphoreType.DMA((2,)),
        ],
    )
    def sc_kernel(x_hbm, gidx_hbm, sidx_hbm, mail1_hbm, mail2_hbm,
                  x_sh, gi_c, si_c, buf1, buf2,
                  sem_i, sem_g1, sem_g2, sem_w1, sem_w2):
        s = lax.axis_index("s")
        w = s * NC + lax.axis_index("c")
        seg = w // 4
        q = w % 4
        base = seg * N
        lo = 31 * q + jnp.minimum(q, 1)
        hi = lo + 32 - jnp.minimum(q, 1)

        @pl.when(s < XLOAD_W)
        def _():
            pltpu.sync_copy(x_hbm.at[pl.ds(s * XLOAD_R, XLOAD_R)],
                            x_sh.at[pl.ds(s * XLOAD_R, XLOAD_R)])

        def fire_idx(i):
            b = lax.rem(i, 4)
            pltpu.async_copy(gidx_hbm.at[seg].at[i], gi_c.at[b], sem_i)
            pltpu.async_copy(sidx_hbm.at[seg].at[i], si_c.at[b], sem_i)

        def wait_idx(i):
            b = lax.rem(i, 4)
            pltpu.make_async_copy(gidx_hbm.at[0].at[0], gi_c.at[b], sem_i).wait()
            pltpu.make_async_copy(gidx_hbm.at[0].at[0], si_c.at[b], sem_i).wait()

        def wait_w(i):
            b = lax.rem(i, 2)
            pltpu.make_async_copy(
                buf1.at[0], mail1_hbm.at[pl.ds(base + i * CH, CH)],
                sem_w1.at[b]).wait()
            pltpu.make_async_copy(
                buf2.at[0], mail2_hbm.at[pl.ds(base + i * CH, CH)],
                sem_w2.at[b]).wait()

        fire_idx(lo)
        plsc.subcore_barrier()

        def body(i, carry):
            b = lax.rem(i, 2)

            @pl.when(i >= lo + 2)
            def _():
                wait_w(i - 2)

            wait_idx(i)

            @pl.when(i + 1 < hi)
            def _():
                fire_idx(i + 1)

            pltpu.async_copy(x_sh.at[gi_c.at[lax.rem(i, 4)]], buf1.at[b], sem_g1)
            pltpu.async_copy(x_sh.at[pl.ds(i * CH, CH)], buf2.at[b], sem_g2)
            pltpu.make_async_copy(x_hbm.at[pl.ds(0, CH)], buf1.at[b], sem_g1).wait()
            pltpu.make_async_copy(x_hbm.at[pl.ds(0, CH)], buf2.at[b], sem_g2).wait()

            pltpu.async_copy(buf1.at[b],
                             mail1_hbm.at[pl.ds(base + i * CH, CH)], sem_w1.at[b])
            pltpu.async_copy(buf2.at[b], mail2_hbm.at[si_c.at[lax.rem(i, 4)]],
                             sem_w2.at[b])
            return carry

        lax.fori_loop(lo, hi, body, 0)
        wait_w(hi - 2)
        wait_w(hi - 1)

    return sc_kernel(x, gidx, sidx)


def _mean_body(x_ref, o_ref):
    i = pl.program_id(0)

    @pl.when(i == 0)
    def _():
        o_ref[...] = jnp.zeros_like(o_ref)

    o_ref[...] += jnp.sum(x_ref[...], axis=0, keepdims=True) * (1.0 / N)


def _tc_mean(x):
    return pl.pallas_call(
        _mean_body,
        grid=(P,),
        in_specs=[pl.BlockSpec((B, D), lambda i: (i, 0))],
        out_specs=pl.BlockSpec((1, D), lambda i: (0, 0)),
        out_shape=jax.ShapeDtypeStruct((1, D), jnp.float32),
    )(x)


def _sig(z):
    return 0.5 * jnp.tanh(0.5 * z) + 0.5


def _cell(m_ref, h_ref, c_ref, w_ref, bi_ref):
    mh = jnp.concatenate([m_ref[0].astype(jnp.bfloat16),
                          h_ref[...].astype(jnp.bfloat16)], axis=1)
    gates = (jnp.dot(mh, w_ref[...], preferred_element_type=jnp.float32)
             + bi_ref[...])
    ii = gates[:, 0:D]
    ff = gates[:, D:2 * D]
    gg = gates[:, 2 * D:3 * D]
    oo = gates[:, 3 * D:4 * D]
    c = _sig(ff) * c_ref[...] + _sig(ii) * jnp.tanh(gg)
    h = _sig(oo) * jnp.tanh(c)
    h_ref[...] = h
    c_ref[...] = c


def _lstm_body_a(m1_ref, m2_ref, wih1_ref, wih2_ref,
                 bi1_ref, bi2_ref,
                 h1o_ref, c1o_ref, h2o_ref, c2o_ref, h1, c1, h2, c2):
    k = pl.program_id(1)

    @pl.when(k == 0)
    def _():
        h1[...] = jnp.zeros_like(h1)
        c1[...] = jnp.zeros_like(c1)
        h2[...] = jnp.zeros_like(h2)
        c2[...] = jnp.zeros_like(c2)

    _cell(m1_ref, h1, c1, wih1_ref, bi1_ref)
    _cell(m2_ref, h2, c2, wih2_ref, bi2_ref)

    @pl.when(k == KSEG - 1)
    def _():
        h1o_ref[...] = h1[...].astype(jnp.bfloat16)
        c1o_ref[...] = c1[...].astype(jnp.bfloat16)
        h2o_ref[...] = h2[...].astype(jnp.bfloat16)
        c2o_ref[...] = c2[...].astype(jnp.bfloat16)


def _lstm_body_m(m1_ref, m2_ref, h1i_ref, c1i_ref, h2i_ref, c2i_ref,
                 wih1_ref, wih2_ref,
                 bi1_ref, bi2_ref,
                 h1o_ref, c1o_ref, h2o_ref, c2o_ref, h1, c1, h2, c2):
    k = pl.program_id(1)

    @pl.when(k == 0)
    def _():
        h1[...] = h1i_ref[...].astype(jnp.float32)
        c1[...] = c1i_ref[...].astype(jnp.float32)
        h2[...] = h2i_ref[...].astype(jnp.float32)
        c2[...] = c2i_ref[...].astype(jnp.float32)

    _cell(m1_ref, h1, c1, wih1_ref, bi1_ref)
    _cell(m2_ref, h2, c2, wih2_ref, bi2_ref)

    @pl.when(k == KSEG - 1)
    def _():
        h1o_ref[...] = h1[...].astype(jnp.bfloat16)
        c1o_ref[...] = c1[...].astype(jnp.bfloat16)
        h2o_ref[...] = h2[...].astype(jnp.bfloat16)
        c2o_ref[...] = c2[...].astype(jnp.bfloat16)


def _lstm_body_b(m1_ref, m2_ref, h1i_ref, c1i_ref, h2i_ref, c2i_ref, x_ref,
                 wih1_ref, wih2_ref,
                 fcn1_ref, fcn2_ref, fcs_ref, bi1_ref, bi2_ref, boff_ref,
                 o_ref, h1, c1, h2, c2):
    k = pl.program_id(1)

    @pl.when(k == 0)
    def _():
        h1[...] = h1i_ref[...].astype(jnp.float32)
        c1[...] = c1i_ref[...].astype(jnp.float32)
        h2[...] = h2i_ref[...].astype(jnp.float32)
        c2[...] = c2i_ref[...].astype(jnp.float32)

    _cell(m1_ref, h1, c1, wih1_ref, bi1_ref)
    _cell(m2_ref, h2, c2, wih2_ref, bi2_ref)

    @pl.when(k == KSEG - 1)
    def _():
        o_ref[...] = (jnp.dot(x_ref[...], fcs_ref[...],
                              preferred_element_type=jnp.float32)
                      + jnp.dot(h1[...], fcn1_ref[...],
                                preferred_element_type=jnp.float32)
                      + jnp.dot(h2[...], fcn2_ref[...],
                                preferred_element_type=jnp.float32)
                      + boff_ref[...])


def _full(shape):
    return pl.BlockSpec(shape, lambda nb, k: tuple(0 for _ in shape))


_MAIL_SPEC = pl.BlockSpec((1, B, D), lambda nb, k: (k, nb, 0))
_ST_SPEC = pl.BlockSpec((B, D), lambda nb, k: (nb, 0))
_MAIL_SPEC_B = pl.BlockSpec((1, BB, D), lambda nb, k: (k, nb, 0))
_ST_SPEC_B = pl.BlockSpec((BB, D), lambda nb, k: (nb, 0))
_STATE = jax.ShapeDtypeStruct((N, D), jnp.bfloat16)


def _tc_lstm_a(mail1, mail2, w1, w2, bi1, bi2):
    return pl.pallas_call(
        _lstm_body_a,
        grid=(P, KSEG),
        in_specs=[
            _MAIL_SPEC, _MAIL_SPEC,
            _full((2 * D, 4 * D)), _full((2 * D, 4 * D)),
            _full((1, 4 * D)), _full((1, 4 * D)),
        ],
        out_specs=[_ST_SPEC, _ST_SPEC, _ST_SPEC, _ST_SPEC],
        out_shape=[_STATE, _STATE, _STATE, _STATE],
        scratch_shapes=[pltpu.VMEM((B, D), jnp.float32)] * 4,
    )(mail1, mail2, w1, w2, bi1, bi2)


def _tc_lstm_m(mail1, mail2, st, w1, w2, bi1, bi2):
    return pl.pallas_call(
        _lstm_body_m,
        grid=(PB, KSEG),
        in_specs=[
            _MAIL_SPEC_B, _MAIL_SPEC_B,
            _ST_SPEC_B, _ST_SPEC_B, _ST_SPEC_B, _ST_SPEC_B,
            _full((2 * D, 4 * D)), _full((2 * D, 4 * D)),
            _full((1, 4 * D)), _full((1, 4 * D)),
        ],
        out_specs=[_ST_SPEC_B, _ST_SPEC_B, _ST_SPEC_B, _ST_SPEC_B],
        out_shape=[_STATE, _STATE, _STATE, _STATE],
        scratch_shapes=[pltpu.VMEM((BB, D), jnp.float32)] * 4,
    )(mail1, mail2, *st, w1, w2, bi1, bi2)


def _tc_lstm_b(mail1, mail2, st, x, w1, w2,
               fcn1T, fcn2T, fcsT, bi1, bi2, boff):
    return pl.pallas_call(
        _lstm_body_b,
        grid=(PB, KSEG),
        in_specs=[
            _MAIL_SPEC_B, _MAIL_SPEC_B,
            _ST_SPEC_B, _ST_SPEC_B, _ST_SPEC_B, _ST_SPEC_B,
            _ST_SPEC_B,
            _full((2 * D, 4 * D)), _full((2 * D, 4 * D)),
            _full((D, D)), _full((D, D)), _full((D, D)),
            _full((1, 4 * D)), _full((1, 4 * D)), _full((1, D)),
        ],
        out_specs=_ST_SPEC_B,
        out_shape=jax.ShapeDtypeStruct((N, D), jnp.float32),
        scratch_shapes=[pltpu.VMEM((BB, D), jnp.float32)] * 4,
    )(mail1, mail2, *st, x, w1, w2,
      fcn1T, fcn2T, fcsT, bi1, bi2, boff)


def kernel(x, edge_index, fc_self1, fc_neigh1, bias1,
           lstm1_Wih, lstm1_Whh, lstm1_bih, lstm1_bhh,
           fc_self2, fc_neigh2, bias2,
           lstm2_Wih, lstm2_Whh, lstm2_bih, lstm2_bhh):
    src = edge_index[0].astype(jnp.int32)
    seg_off = jnp.repeat(jnp.arange(DEG, dtype=jnp.int32) % KSEG * N, N)
    gidx = src.reshape(DEG, NCH, CH)
    sidx = (src + seg_off).reshape(DEG, NCH, CH)

    xmean = _tc_mean(x)

    w1 = jnp.concatenate([lstm1_Wih.T, lstm1_Whh.T],
                         axis=0).astype(jnp.bfloat16)
    w2 = jnp.concatenate([lstm2_Wih.T, lstm2_Whh.T],
                         axis=0).astype(jnp.bfloat16)
    bi1 = (lstm1_bih + lstm1_bhh).reshape(1, 4 * D)
    bi2 = (lstm2_bih + lstm2_bhh).reshape(1, 4 * D)
    fcn1T = fc_neigh1.T
    fcn2T = fc_neigh2.T
    fcsT = (fc_self1 + fc_self2).T
    boff = (bias1 + bias2).reshape(1, D) + xmean

    mails = [
        _sc_build_mailboxes(x, gidx[i * KSEG:(i + 1) * KSEG],
                            sidx[i * KSEG:(i + 1) * KSEG])
        for i in range(4)
    ]
    mails = [(a.reshape(KSEG, N, D), b.reshape(KSEG, N, D)) for a, b in mails]

    st = _tc_lstm_a(*mails[0], w1, w2, bi1, bi2)
    st = _tc_lstm_m(*mails[1], st, w1, w2, bi1, bi2)
    st = _tc_lstm_m(*mails[2], st, w1, w2, bi1, bi2)

    return _tc_lstm_b(*mails[3], st, x, w1, w2,
                      fcn1T, fcn2T, fcsT, bi1, bi2, boff)
